# baseline (speedup 1.0000x reference)
import functools

import jax
import jax.numpy as jnp
from jax import lax
from jax.experimental import pallas as pl
from jax.experimental.pallas import tpu as pltpu, tpu_sc as plsc

K = 3
N = 4096
NS = 16384
D = 256
DS = 128
H = 256
B = 512
BM = 2048

NSP = 2048
NW = 32
QW = NSP // NW
C = 32
NCHUNK = QW // C


def _topk_body(posq, posct, idx_out, wexp_out):
    q = posq[:]
    ct = posct[:]

    qn = jnp.sum(q * q, axis=1, keepdims=True)
    cn = jnp.sum(ct * ct, axis=0, keepdims=True)
    qc = lax.dot_general(q.astype(jnp.bfloat16), ct.astype(jnp.bfloat16),
                         (((1,), (0,)), ((), ())),
                         preferred_element_type=jnp.float32)
    d2 = jnp.maximum(qn + cn - 2.0 * qc, 0.0)

    iota = lax.broadcasted_iota(jnp.int32, (B, N), 1)
    big = jnp.int32(2**30)
    inf = jnp.float32(jnp.inf)

    js = []
    ws = []
    wsum = jnp.zeros((B, 1), jnp.float32)
    for k in range(K):
        m = jnp.min(d2, axis=1, keepdims=True)
        j = jnp.min(jnp.where(d2 == m, iota, big), axis=1, keepdims=True)
        wk = 1.0 / jnp.maximum(m, 1e-16)
        js.append(j)
        ws.append(wk)
        wsum = wsum + wk
        if k < K - 1:
            d2 = jnp.where(iota == j, inf, d2)

    idx_out[:] = jnp.concatenate(js, axis=1)
    wexp_out[:] = jnp.concatenate(
        [jnp.broadcast_to(w / wsum, (B, 16)) for w in ws], axis=1)


@jax.jit
def _topk(posq_pad, posct_pad):
    return pl.pallas_call(
        _topk_body,
        grid=(NSP // B,),
        in_specs=[
            pl.BlockSpec((B, 8), lambda i: (i, 0)),
            pl.BlockSpec((8, N), lambda i: (0, 0)),
        ],
        out_specs=[
            pl.BlockSpec((B, K), lambda i: (i, 0)),
            pl.BlockSpec((B, K * 16), lambda i: (i, 0)),
        ],
        out_shape=[
            jax.ShapeDtypeStruct((NSP, K), jnp.int32),
            jax.ShapeDtypeStruct((NSP, K * 16), jnp.float32),
        ],
    )(posq_pad, posct_pad)


def _sc_body(x_hbm, idx_hbm, w_hbm, h_hbm, idx_v, w_v, rows_v, h_v, sem):
    wid = lax.axis_index("s") * 2 + lax.axis_index("c")
    qbase0 = wid * QW

    def chunk(c, _):
        qbase = qbase0 + c * C
        pltpu.sync_copy(idx_hbm.at[pl.ds(qbase * K, C * K)], idx_v)
        pltpu.sync_copy(w_hbm.at[pl.ds(qbase * K * 16, C * K * 16)], w_v)
        pltpu.async_copy(x_hbm.at[idx_v], rows_v, sem).wait()

        def q_iter(qi, _):
            wv = [w_v[pl.ds(qi * (K * 16) + k * 16, 16)] for k in range(K)]
            for f in range(D // 16):
                acc = wv[0] * rows_v[qi * K, pl.ds(f * 16, 16)]
                acc = acc + wv[1] * rows_v[qi * K + 1, pl.ds(f * 16, 16)]
                acc = acc + wv[2] * rows_v[qi * K + 2, pl.ds(f * 16, 16)]
                h_v[qi, pl.ds(f * 16, 16)] = acc
            return 0

        lax.fori_loop(0, C, q_iter, 0)
        pltpu.sync_copy(h_v, h_hbm.at[pl.ds(qbase, C)])
        return 0

    lax.fori_loop(0, NCHUNK, chunk, 0)


@jax.jit
def _sc_gather(x, idx_flat, w_flat):
    mesh = plsc.VectorSubcoreMesh(core_axis_name="c", subcore_axis_name="s")
    f = pl.kernel(
        _sc_body,
        out_type=jax.ShapeDtypeStruct((NSP, D), jnp.float32),
        mesh=mesh,
        scratch_types=[
            pltpu.VMEM((C * K,), jnp.int32),
            pltpu.VMEM((C * K * 16,), jnp.float32),
            pltpu.VMEM((C * K, D), jnp.float32),
            pltpu.VMEM((C, D), jnp.float32),
            pltpu.SemaphoreType.DMA,
        ],
    )
    return f(x, idx_flat, w_flat)


def _mlp_body(hb, xs, w1h, w1s, b1, w2, b2, out):
    a = (lax.dot_general(hb[:].astype(jnp.bfloat16), w1h[:],
                         (((1,), (0,)), ((), ())),
                         preferred_element_type=jnp.float32)
         + lax.dot_general(xs[:], w1s[:], (((1,), (0,)), ((), ())),
                           preferred_element_type=jnp.float32)
         + b1[:])
    a = jnp.maximum(a, 0.0)
    out[:] = lax.dot_general(a.astype(jnp.bfloat16), w2[:],
                             (((1,), (0,)), ((), ())),
                             preferred_element_type=jnp.float32) + b2[:]


@jax.jit
def _mlp(h, xsb, W1h, W1s, b1, W2, b2):
    return pl.pallas_call(
        _mlp_body,
        grid=(NSP // BM,),
        in_specs=[
            pl.BlockSpec((BM, D), lambda i: (i, 0)),
            pl.BlockSpec((BM, DS), lambda i: (i, 0)),
            pl.BlockSpec((D, H), lambda i: (0, 0)),
            pl.BlockSpec((DS, H), lambda i: (0, 0)),
            pl.BlockSpec((1, H), lambda i: (0, 0)),
            pl.BlockSpec((H, H), lambda i: (0, 0)),
            pl.BlockSpec((1, H), lambda i: (0, 0)),
        ],
        out_specs=pl.BlockSpec((BM, H), lambda i: (i, 0)),
        out_shape=jax.ShapeDtypeStruct((NSP, H), jnp.float32),
    )(h, xsb, W1h, W1s, b1, W2, b2)


def kernel(x, pos, batch, x_skip, pos_skip, batch_skip, W1, b1, W2, b2):
    posq_pad = jnp.zeros((NS, 8), jnp.float32).at[:, :3].set(pos_skip)
    posct_pad = jnp.zeros((8, N), jnp.float32).at[:3, :].set(pos.T)
    xsb = x_skip.astype(jnp.bfloat16)
    W1h = W1[:D].astype(jnp.bfloat16)
    W1s = W1[D:].astype(jnp.bfloat16)
    W2b = W2.astype(jnp.bfloat16)
    b1r = b1.reshape(1, H)
    b2r = b2.reshape(1, H)
    outs = []
    for s in range(NS // NSP):
        lo = s * NSP
        idx, wexp = _topk(posq_pad[lo:lo + NSP], posct_pad)
        h = _sc_gather(x, idx.reshape(NSP * K), wexp.reshape(NSP * K * 16))
        outs.append(_mlp(h, xsb[lo:lo + NSP], W1h, W1s, b1r, W2b, b2r))
    return (jnp.concatenate(outs, axis=0), pos_skip, batch_skip)


# speedup vs baseline: 1.2184x; 1.2184x over previous
import functools

import jax
import jax.numpy as jnp
from jax import lax
from jax.experimental import pallas as pl
from jax.experimental.pallas import tpu as pltpu, tpu_sc as plsc

K = 3
N = 4096
NS = 16384
D = 256
DS = 128
H = 256
B = 512
BM = 1024

NSP = 4096
NW = 32
QW = NSP // NW
C = 32
NCHUNK = QW // C


def _topk_body(posq, posct, idx_out, wexp_out):
    q = posq[:]
    ct = posct[:]

    qn = jnp.sum(q * q, axis=1, keepdims=True)
    cn = jnp.sum(ct * ct, axis=0, keepdims=True)
    qc = lax.dot_general(q.astype(jnp.bfloat16), ct.astype(jnp.bfloat16),
                         (((1,), (0,)), ((), ())),
                         preferred_element_type=jnp.float32)
    d2 = jnp.maximum(qn + cn - 2.0 * qc, 0.0)

    iota = lax.broadcasted_iota(jnp.int32, (B, N), 1).astype(jnp.float32)
    big = jnp.float32(1e9)
    inf = jnp.float32(jnp.inf)

    js = []
    ws = []
    wsum = jnp.zeros((B, 1), jnp.float32)
    for k in range(K):
        m = jnp.min(d2, axis=1, keepdims=True)
        j = jnp.min(jnp.where(d2 == m, iota, big), axis=1, keepdims=True)
        wk = 1.0 / jnp.maximum(m, 1e-16)
        js.append(j)
        ws.append(wk)
        wsum = wsum + wk
        if k < K - 1:
            d2 = jnp.where(iota == j, inf, d2)

    idx_out[:] = jnp.concatenate(js, axis=1).astype(jnp.int32)
    wexp_out[:] = jnp.concatenate(
        [jnp.broadcast_to(w / wsum, (B, 16)) for w in ws], axis=1)


@jax.jit
def _topk(posq_pad, posct_pad):
    return pl.pallas_call(
        _topk_body,
        grid=(NSP // B,),
        in_specs=[
            pl.BlockSpec((B, 8), lambda i: (i, 0)),
            pl.BlockSpec((8, N), lambda i: (0, 0)),
        ],
        out_specs=[
            pl.BlockSpec((B, K), lambda i: (i, 0)),
            pl.BlockSpec((B, K * 16), lambda i: (i, 0)),
        ],
        out_shape=[
            jax.ShapeDtypeStruct((NSP, K), jnp.int32),
            jax.ShapeDtypeStruct((NSP, K * 16), jnp.float32),
        ],
    )(posq_pad, posct_pad)


def _sc_body(x_hbm, idx_hbm, w_hbm, h_hbm, idx_v, w_v, rows_v, h_v, sem):
    wid = lax.axis_index("s") * 2 + lax.axis_index("c")
    qbase0 = wid * QW

    def chunk(c, _):
        qbase = qbase0 + c * C
        pltpu.sync_copy(idx_hbm.at[pl.ds(qbase * K, C * K)], idx_v)
        pltpu.sync_copy(w_hbm.at[pl.ds(qbase * K * 16, C * K * 16)], w_v)
        pltpu.async_copy(x_hbm.at[idx_v], rows_v, sem).wait()

        def q_iter(qi, _):
            wv = [w_v[pl.ds(qi * (K * 16) + k * 16, 16)] for k in range(K)]
            for f in range(D // 16):
                acc = wv[0] * rows_v[qi * K, pl.ds(f * 16, 16)]
                acc = acc + wv[1] * rows_v[qi * K + 1, pl.ds(f * 16, 16)]
                acc = acc + wv[2] * rows_v[qi * K + 2, pl.ds(f * 16, 16)]
                h_v[qi, pl.ds(f * 16, 16)] = acc
            return 0

        lax.fori_loop(0, C, q_iter, 0)
        pltpu.sync_copy(h_v, h_hbm.at[pl.ds(qbase, C)])
        return 0

    lax.fori_loop(0, NCHUNK, chunk, 0)


@jax.jit
def _sc_gather(x, idx_flat, w_flat):
    mesh = plsc.VectorSubcoreMesh(core_axis_name="c", subcore_axis_name="s")
    f = pl.kernel(
        _sc_body,
        out_type=jax.ShapeDtypeStruct((NSP, D), jnp.float32),
        mesh=mesh,
        scratch_types=[
            pltpu.VMEM((C * K,), jnp.int32),
            pltpu.VMEM((C * K * 16,), jnp.float32),
            pltpu.VMEM((C * K, D), jnp.float32),
            pltpu.VMEM((C, D), jnp.float32),
            pltpu.SemaphoreType.DMA,
        ],
    )
    return f(x, idx_flat, w_flat)


def _mlp_body(hb, xs, w1h, w1s, b1, w2, b2, out):
    a = (lax.dot_general(hb[:].astype(jnp.bfloat16), w1h[:],
                         (((1,), (0,)), ((), ())),
                         preferred_element_type=jnp.float32)
         + lax.dot_general(xs[:], w1s[:], (((1,), (0,)), ((), ())),
                           preferred_element_type=jnp.float32)
         + b1[:])
    a = jnp.maximum(a, 0.0)
    out[:] = lax.dot_general(a.astype(jnp.bfloat16), w2[:],
                             (((1,), (0,)), ((), ())),
                             preferred_element_type=jnp.float32) + b2[:]


@jax.jit
def _mlp(h, xsb, W1h, W1s, b1, W2, b2):
    return pl.pallas_call(
        _mlp_body,
        grid=(NSP // BM,),
        in_specs=[
            pl.BlockSpec((BM, D), lambda i: (i, 0)),
            pl.BlockSpec((BM, DS), lambda i: (i, 0)),
            pl.BlockSpec((D, H), lambda i: (0, 0)),
            pl.BlockSpec((DS, H), lambda i: (0, 0)),
            pl.BlockSpec((1, H), lambda i: (0, 0)),
            pl.BlockSpec((H, H), lambda i: (0, 0)),
            pl.BlockSpec((1, H), lambda i: (0, 0)),
        ],
        out_specs=pl.BlockSpec((BM, H), lambda i: (i, 0)),
        out_shape=jax.ShapeDtypeStruct((NSP, H), jnp.float32),
    )(h, xsb, W1h, W1s, b1, W2, b2)


def kernel(x, pos, batch, x_skip, pos_skip, batch_skip, W1, b1, W2, b2):
    posq_pad = jnp.zeros((NS, 8), jnp.float32).at[:, :3].set(pos_skip)
    posct_pad = jnp.zeros((8, N), jnp.float32).at[:3, :].set(pos.T)
    xsb = x_skip.astype(jnp.bfloat16)
    W1h = W1[:D].astype(jnp.bfloat16)
    W1s = W1[D:].astype(jnp.bfloat16)
    W2b = W2.astype(jnp.bfloat16)
    b1r = b1.reshape(1, H)
    b2r = b2.reshape(1, H)
    outs = []
    for s in range(NS // NSP):
        lo = s * NSP
        idx, wexp = _topk(posq_pad[lo:lo + NSP], posct_pad)
        h = _sc_gather(x, idx.reshape(NSP * K), wexp.reshape(NSP * K * 16))
        outs.append(_mlp(h, xsb[lo:lo + NSP], W1h, W1s, b1r, W2b, b2r))
    return (jnp.concatenate(outs, axis=0), pos_skip, batch_skip)
